# baseline (device time: 14156 ns/iter reference)
import jax
import jax.numpy as jnp
from jax import lax
from jax.experimental import pallas as pl
from jax.experimental.pallas import tpu as pltpu

B, SQ, SKV, H, D = 8, 1, 512, 8, 64


def kernel(Q, K, V):
    scale = D ** -0.5
    Kt = jnp.transpose(K, (0, 2, 3, 1))
    Vt = jnp.transpose(V, (0, 2, 3, 1))

    def body(q_ref, k_ref, v_ref, out_ref, kbuf, vbuf,
             send_buf, recv_buf, obuf, sems, send_sem, recv_sem, out_sem):
        my_x = lax.axis_index("x")
        my_y = lax.axis_index("y")
        my_z = lax.axis_index("z")
        nbr = (my_x, 1 - my_y, my_z)

        barrier_sem = pltpu.get_barrier_semaphore()
        pl.semaphore_signal(
            barrier_sem, inc=1, device_id=nbr, device_id_type=pl.DeviceIdType.MESH
        )

        copies = []
        for b in range(B):
            kc = pltpu.make_async_copy(k_ref.at[b], kbuf.at[b], sems.at[b])
            vc = pltpu.make_async_copy(v_ref.at[b], vbuf.at[b], sems.at[B + b])
            kc.start()
            vc.start()
            copies.append((kc, vc))

        for b in range(B):
            kc, vc = copies[b]
            q_b = q_ref[b, 0]
            kc.wait()
            s = jnp.sum(q_b[:, :, None] * kbuf[b], axis=1) * scale
            m = jnp.max(s, axis=-1, keepdims=True)
            p = jnp.exp(s - m)
            l = jnp.sum(p, axis=-1, keepdims=True)
            vc.wait()
            o = jnp.sum(p[:, None, :] * vbuf[b], axis=-1)
            send_buf[b, :, 0:D] = o
            send_buf[b, :, D:2 * D] = jnp.broadcast_to(m, (H, D))
            send_buf[b, :, 2 * D:3 * D] = jnp.broadcast_to(l, (H, D))

        pl.semaphore_wait(barrier_sem, 1)
        rdma = pltpu.make_async_remote_copy(
            src_ref=send_buf,
            dst_ref=recv_buf,
            send_sem=send_sem,
            recv_sem=recv_sem,
            device_id=nbr,
            device_id_type=pl.DeviceIdType.MESH,
        )
        rdma.start()
        rdma.wait()

        o1 = send_buf[:, :, 0:D]
        m1 = send_buf[:, :, D:2 * D]
        l1 = send_buf[:, :, 2 * D:3 * D]
        o2 = recv_buf[:, :, 0:D]
        m2 = recv_buf[:, :, D:2 * D]
        l2 = recv_buf[:, :, 2 * D:3 * D]
        mn = jnp.maximum(m1, m2)
        a1 = jnp.exp(m1 - mn)
        a2 = jnp.exp(m2 - mn)
        obuf[:, 0, :, :] = (a1 * o1 + a2 * o2) / (a1 * l1 + a2 * l2)
        out_copy = pltpu.make_async_copy(obuf, out_ref, out_sem)
        out_copy.start()
        out_copy.wait()

    return pl.pallas_call(
        body,
        out_shape=jax.ShapeDtypeStruct((B, SQ, H, D), jnp.float32),
        in_specs=[
            pl.BlockSpec(memory_space=pltpu.MemorySpace.VMEM),
            pl.BlockSpec(memory_space=pl.ANY),
            pl.BlockSpec(memory_space=pl.ANY),
        ],
        out_specs=pl.BlockSpec(memory_space=pl.ANY),
        scratch_shapes=[
            pltpu.VMEM((B, H, D, SKV), jnp.float32),
            pltpu.VMEM((B, H, D, SKV), jnp.float32),
            pltpu.VMEM((B, H, 3 * D), jnp.float32),
            pltpu.VMEM((B, H, 3 * D), jnp.float32),
            pltpu.VMEM((B, SQ, H, D), jnp.float32),
            pltpu.SemaphoreType.DMA((2 * B,)),
            pltpu.SemaphoreType.DMA,
            pltpu.SemaphoreType.DMA,
            pltpu.SemaphoreType.DMA,
        ],
        compiler_params=pltpu.CompilerParams(
            collective_id=0,
            vmem_limit_bytes=96 * 1024 * 1024,
        ),
    )(Q, Kt, Vt)


# device time: 13874 ns/iter; 1.0203x vs baseline; 1.0203x over previous
import jax
import jax.numpy as jnp
from jax import lax
from jax.experimental import pallas as pl
from jax.experimental.pallas import tpu as pltpu

B, SQ, SKV, H, D = 8, 1, 512, 8, 64


def kernel(Q, K, V):
    scale = D ** -0.5
    Kt = jnp.transpose(K, (0, 2, 3, 1))
    Vt = jnp.transpose(V, (0, 2, 3, 1))

    def body(q_ref, k_ref, v_ref, out_ref, kbuf, vbuf,
             send_buf, recv_buf, obuf, sems, send_sems, recv_sems, out_sem):
        my_x = lax.axis_index("x")
        my_y = lax.axis_index("y")
        my_z = lax.axis_index("z")
        nbr = (my_x, 1 - my_y, my_z)

        barrier_sem = pltpu.get_barrier_semaphore()
        pl.semaphore_signal(
            barrier_sem, inc=1, device_id=nbr, device_id_type=pl.DeviceIdType.MESH
        )

        copies = []
        for b in range(B):
            kc = pltpu.make_async_copy(k_ref.at[b], kbuf.at[b], sems.at[b])
            vc = pltpu.make_async_copy(v_ref.at[b], vbuf.at[b], sems.at[B + b])
            kc.start()
            vc.start()
            copies.append((kc, vc))

        rdmas = []
        for b in range(B):
            kc, vc = copies[b]
            q_b = q_ref[b, 0]
            kc.wait()
            s = lax.dot_general(
                q_b, kbuf[b],
                (((1,), (1,)), ((0,), (0,))),
                preferred_element_type=jnp.float32,
            ) * scale
            m = jnp.max(s, axis=-1, keepdims=True)
            p = jnp.exp(s - m)
            l = jnp.sum(p, axis=-1, keepdims=True)
            vc.wait()
            o = lax.dot_general(
                p, vbuf[b],
                (((1,), (2,)), ((0,), (0,))),
                preferred_element_type=jnp.float32,
            )
            send_buf[b, :, 0:D] = o
            send_buf[b, :, D:2 * D] = jnp.broadcast_to(m, (H, D))
            send_buf[b, :, 2 * D:3 * D] = jnp.broadcast_to(l, (H, D))
            if b == 0:
                pl.semaphore_wait(barrier_sem, 1)
            rdma = pltpu.make_async_remote_copy(
                src_ref=send_buf.at[b],
                dst_ref=recv_buf.at[b],
                send_sem=send_sems.at[b],
                recv_sem=recv_sems.at[b],
                device_id=nbr,
                device_id_type=pl.DeviceIdType.MESH,
            )
            rdma.start()
            rdmas.append(rdma)

        for b in range(B):
            rdmas[b].wait_send()
            rdmas[b].wait_recv()
        o1 = send_buf[:, :, 0:D]
        m1 = send_buf[:, :, D:2 * D]
        l1 = send_buf[:, :, 2 * D:3 * D]
        o2 = recv_buf[:, :, 0:D]
        m2 = recv_buf[:, :, D:2 * D]
        l2 = recv_buf[:, :, 2 * D:3 * D]
        mn = jnp.maximum(m1, m2)
        a1 = jnp.exp(m1 - mn)
        a2 = jnp.exp(m2 - mn)
        obuf[:, 0, :, :] = (a1 * o1 + a2 * o2) / (a1 * l1 + a2 * l2)
        out_copy = pltpu.make_async_copy(obuf, out_ref, out_sem)
        out_copy.start()
        out_copy.wait()

    return pl.pallas_call(
        body,
        out_shape=jax.ShapeDtypeStruct((B, SQ, H, D), jnp.float32),
        in_specs=[
            pl.BlockSpec(memory_space=pltpu.MemorySpace.VMEM),
            pl.BlockSpec(memory_space=pl.ANY),
            pl.BlockSpec(memory_space=pl.ANY),
        ],
        out_specs=pl.BlockSpec(memory_space=pl.ANY),
        scratch_shapes=[
            pltpu.VMEM((B, H, D, SKV), jnp.float32),
            pltpu.VMEM((B, H, D, SKV), jnp.float32),
            pltpu.VMEM((B, H, 3 * D), jnp.float32),
            pltpu.VMEM((B, H, 3 * D), jnp.float32),
            pltpu.VMEM((B, SQ, H, D), jnp.float32),
            pltpu.SemaphoreType.DMA((2 * B,)),
            pltpu.SemaphoreType.DMA((B,)),
            pltpu.SemaphoreType.DMA((B,)),
            pltpu.SemaphoreType.DMA,
        ],
        compiler_params=pltpu.CompilerParams(
            collective_id=0,
            vmem_limit_bytes=96 * 1024 * 1024,
        ),
    )(Q, Kt, Vt)
